# baseline (speedup 1.0000x reference)
import functools

import jax
import jax.numpy as jnp
from jax import lax
from jax.experimental import pallas as pl
from jax.experimental.pallas import tpu as pltpu
from jax.experimental.pallas import tpu_sc as plsc

N_CTX = 2048
N_EMBD = 768
N_EXP = 64
B = 4
FLAT = N_CTX * N_EMBD
LANES = 16

S_TC = 1792
N_WORKERS = 32
S_W = (N_CTX - S_TC) // N_WORKERS
F_W = S_W * N_EMBD
E_G = 8
N_GROUPS = N_EXP // E_G
F_H = F_W // 2
R_H = F_H // N_EMBD

CH_S = 64
CHUNK = CH_S * N_EMBD
N_STEPS = S_TC // CH_S


def _matmul_kernel(x_ref, w_ref, o_ref, acc_ref):
    i = pl.program_id(0)
    part = jnp.zeros((B, N_EXP), jnp.float32)
    for s in range(CH_S):
        part = part + jax.lax.dot_general(
            x_ref[:, s, :], w_ref[:, s * N_EMBD:(s + 1) * N_EMBD],
            dimension_numbers=(((1,), (1,)), ((), ())),
            preferred_element_type=jnp.float32,
        )

    @pl.when(i == 0)
    def _init():
        acc_ref[...] = part

    @pl.when(i > 0)
    def _acc():
        acc_ref[...] = acc_ref[...] + part

    @pl.when(i == N_STEPS - 1)
    def _flush():
        o_ref[...] = acc_ref[...]


@functools.partial(
    pl.kernel,
    out_type=jax.ShapeDtypeStruct((N_WORKERS, B, N_EXP, LANES), jnp.float32),
    mesh=plsc.VectorSubcoreMesh(core_axis_name="c", subcore_axis_name="s"),
    scratch_types=[
        pltpu.VMEM((B, S_W, N_EMBD), jnp.float32),
        pltpu.VMEM((E_G, F_H), jnp.float32),
        pltpu.VMEM((E_G, F_H), jnp.float32),
        pltpu.VMEM((B, N_EXP, LANES), jnp.float32),
        pltpu.SemaphoreType.DMA,
        pltpu.SemaphoreType.DMA,
    ],
)
def _sc_partial(x_hbm, w_hbm, out_hbm, xbuf, wbuf0, wbuf1, obuf, sem0, sem1):
    wid = lax.axis_index("s") * 2 + lax.axis_index("c")
    row0 = S_TC + wid * S_W
    col0 = row0 * N_EMBD
    wbufs = (wbuf0, wbuf1)
    sems = (sem0, sem1)

    pltpu.sync_copy(x_hbm.at[:, pl.ds(row0, S_W), :], xbuf)

    def wcopy(k, buf, sem):
        g, h = k // 2, k % 2
        return pltpu.async_copy(
            w_hbm.at[pl.ds(g * E_G, E_G), pl.ds(col0 + h * F_H, F_H)],
            buf, sem,
        )

    n_tiles = N_GROUPS * 2
    pending = wcopy(0, wbuf0, sem0)
    for g in range(N_GROUPS):
        accs = tuple(jnp.zeros((LANES,), jnp.float32)
                     for _ in range(E_G * B))
        for h in range(2):
            k = g * 2 + h
            if k + 1 < n_tiles:
                nxt = wcopy(k + 1, wbufs[(k + 1) % 2], sems[(k + 1) % 2])
            pending.wait()
            wbuf = wbufs[k % 2]

            for rl in range(R_H):
                r = h * R_H + rl

                def body(t, accs, r=r, rl=rl, wbuf=wbuf):
                    o = t * LANES
                    xv = [xbuf[bb, r, pl.ds(o, LANES)] for bb in range(B)]
                    out = []
                    for e in range(E_G):
                        wv = wbuf[e, pl.ds(rl * N_EMBD + o, LANES)]
                        for bb in range(B):
                            out.append(accs[e * B + bb] + wv * xv[bb])
                    return tuple(out)

                accs = lax.fori_loop(0, N_EMBD // LANES, body, accs)
            if k + 1 < n_tiles:
                pending = nxt
        for e in range(E_G):
            for bb in range(B):
                obuf[bb, g * E_G + e, :] = accs[e * B + bb]

    pltpu.sync_copy(obuf, out_hbm.at[wid])


def _route_kernel(ytc_ref, sc_ref, b_ref, o_ref):
    y = ytc_ref[...] + jnp.sum(sc_ref[...], axis=(0, 3)) + b_ref[...]
    col = jax.lax.broadcasted_iota(jnp.int32, (B, N_EXP), 1)
    v1 = jnp.max(y, axis=1, keepdims=True)
    i1 = jnp.min(jnp.where(y == v1, col, N_EXP), axis=1, keepdims=True)
    sel1 = col == i1
    y2 = jnp.where(sel1, -jnp.inf, y)
    v2 = jnp.max(y2, axis=1, keepdims=True)
    i2 = jnp.min(jnp.where(y2 == v2, col, N_EXP), axis=1, keepdims=True)
    sel2 = col == i2
    mask = jnp.where(sel1 | sel2, y, 0.0)
    m = jnp.max(mask, axis=1, keepdims=True)
    e = jnp.exp(mask - m)
    o_ref[...] = e / jnp.sum(e, axis=1, keepdims=True)


@jax.jit
def kernel(x, W, b):
    b2 = b.reshape(1, N_EXP)
    y_sc = _sc_partial(x, W)
    y_tc = pl.pallas_call(
        _matmul_kernel,
        grid=(N_STEPS,),
        in_specs=[
            pl.BlockSpec((B, CH_S, N_EMBD), lambda i: (0, i, 0)),
            pl.BlockSpec((N_EXP, CHUNK), lambda i: (0, i)),
        ],
        out_specs=pl.BlockSpec((B, N_EXP), lambda i: (0, 0)),
        out_shape=jax.ShapeDtypeStruct((B, N_EXP), jnp.float32),
        scratch_shapes=[pltpu.VMEM((B, N_EXP), jnp.float32)],
    )(x, W)
    return pl.pallas_call(
        _route_kernel,
        in_specs=[
            pl.BlockSpec((B, N_EXP), lambda: (0, 0)),
            pl.BlockSpec((N_WORKERS, B, N_EXP, LANES), lambda: (0, 0, 0, 0)),
            pl.BlockSpec((1, N_EXP), lambda: (0, 0)),
        ],
        out_specs=pl.BlockSpec((B, N_EXP), lambda: (0, 0)),
        out_shape=jax.ShapeDtypeStruct((B, N_EXP), jnp.float32),
    )(y_tc, y_sc, b2)


# speedup vs baseline: 1.1687x; 1.1687x over previous
import jax
import jax.numpy as jnp
from jax.experimental import pallas as pl
from jax.experimental.pallas import tpu as pltpu

N_CTX = 2048
N_EMBD = 768
N_EXP = 64
B = 4
FLAT = N_CTX * N_EMBD
HALF = N_EXP // 2

CH_S = 64
CHUNK = CH_S * N_EMBD
N_STEPS = N_CTX // CH_S


def _router_kernel(x_ref, wt_ref, wb_ref, b_ref, o_ref, acc_ref):
    i = pl.program_id(0)
    dn = (((1,), (1,)), ((), ()))
    pt = jnp.zeros((B, HALF), jnp.float32)
    pb = jnp.zeros((B, HALF), jnp.float32)
    for s in range(CH_S):
        xs = x_ref[:, s, :]
        ws = slice(s * N_EMBD, (s + 1) * N_EMBD)
        pt = pt + jax.lax.dot_general(
            xs, wt_ref[:, ws], dimension_numbers=dn,
            preferred_element_type=jnp.float32)
        pb = pb + jax.lax.dot_general(
            xs, wb_ref[:, ws], dimension_numbers=dn,
            preferred_element_type=jnp.float32)
    part = jnp.concatenate([pt, pb], axis=1)

    @pl.when(i == 0)
    def _init():
        acc_ref[...] = part

    @pl.when(i > 0)
    def _acc():
        acc_ref[...] = acc_ref[...] + part

    @pl.when(i == N_STEPS - 1)
    def _epilogue():
        y = acc_ref[...] + b_ref[...]
        col = jax.lax.broadcasted_iota(jnp.int32, (B, N_EXP), 1)
        v1 = jnp.max(y, axis=1, keepdims=True)
        i1 = jnp.min(jnp.where(y == v1, col, N_EXP), axis=1, keepdims=True)
        sel1 = col == i1
        y2 = jnp.where(sel1, -jnp.inf, y)
        v2 = jnp.max(y2, axis=1, keepdims=True)
        i2 = jnp.min(jnp.where(y2 == v2, col, N_EXP), axis=1, keepdims=True)
        sel2 = col == i2
        mask = jnp.where(sel1 | sel2, y, 0.0)
        m = jnp.max(mask, axis=1, keepdims=True)
        e = jnp.exp(mask - m)
        o_ref[...] = e / jnp.sum(e, axis=1, keepdims=True)


@jax.jit
def kernel(x, W, b):
    b2 = b.reshape(1, N_EXP)
    return pl.pallas_call(
        _router_kernel,
        grid=(N_STEPS,),
        in_specs=[
            pl.BlockSpec((B, CH_S, N_EMBD), lambda i: (0, i, 0)),
            pl.BlockSpec((HALF, CHUNK), lambda i: (0, i)),
            pl.BlockSpec((HALF, CHUNK), lambda i: (1, i)),
            pl.BlockSpec((1, N_EXP), lambda i: (0, 0)),
        ],
        out_specs=pl.BlockSpec((B, N_EXP), lambda i: (0, 0)),
        out_shape=jax.ShapeDtypeStruct((B, N_EXP), jnp.float32),
        scratch_shapes=[pltpu.VMEM((B, N_EXP), jnp.float32)],
    )(x, W, W, b2)


# speedup vs baseline: 1.1695x; 1.0007x over previous
import jax
import jax.numpy as jnp
from jax.experimental import pallas as pl
from jax.experimental.pallas import tpu as pltpu

N_CTX = 2048
N_EMBD = 768
N_EXP = 64
B = 4
FLAT = N_CTX * N_EMBD

CH_S = 64
CHUNK = CH_S * N_EMBD
N_STEPS = N_CTX // CH_S


def _router_kernel(x_ref, w_ref, b_ref, o_ref, acc_ref):
    i = pl.program_id(0)
    part = jnp.zeros((B, N_EXP), jnp.float32)
    for s in range(CH_S):
        part = part + jax.lax.dot_general(
            x_ref[:, s, :], w_ref[:, s * N_EMBD:(s + 1) * N_EMBD],
            dimension_numbers=(((1,), (1,)), ((), ())),
            preferred_element_type=jnp.float32,
        )

    @pl.when(i == 0)
    def _init():
        acc_ref[...] = part

    @pl.when(i > 0)
    def _acc():
        acc_ref[...] = acc_ref[...] + part

    @pl.when(i == N_STEPS - 1)
    def _epilogue():
        y = acc_ref[...] + b_ref[...]
        col = jax.lax.broadcasted_iota(jnp.int32, (B, N_EXP), 1)
        v1 = jnp.max(y, axis=1, keepdims=True)
        i1 = jnp.min(jnp.where(y == v1, col, N_EXP), axis=1, keepdims=True)
        sel1 = col == i1
        y2 = jnp.where(sel1, -jnp.inf, y)
        v2 = jnp.max(y2, axis=1, keepdims=True)
        i2 = jnp.min(jnp.where(y2 == v2, col, N_EXP), axis=1, keepdims=True)
        sel2 = col == i2
        mask = jnp.where(sel1 | sel2, y, 0.0)
        m = jnp.max(mask, axis=1, keepdims=True)
        e = jnp.exp(mask - m)
        o_ref[...] = e / jnp.sum(e, axis=1, keepdims=True)


@jax.jit
def kernel(x, W, b):
    b2 = b.reshape(1, N_EXP)
    return pl.pallas_call(
        _router_kernel,
        grid=(N_STEPS,),
        in_specs=[
            pl.BlockSpec((B, CH_S, N_EMBD), lambda i: (0, i, 0)),
            pl.BlockSpec((N_EXP, CHUNK), lambda i: (0, i)),
            pl.BlockSpec((1, N_EXP), lambda i: (0, 0)),
        ],
        out_specs=pl.BlockSpec((B, N_EXP), lambda i: (0, 0)),
        out_shape=jax.ShapeDtypeStruct((B, N_EXP), jnp.float32),
        scratch_shapes=[pltpu.VMEM((B, N_EXP), jnp.float32)],
    )(x, W, b2)
